# Initial kernel scaffold; baseline (speedup 1.0000x reference)
import jax
import jax.numpy as jnp
from jax.experimental import pallas as pl


def kernel(x, W3, W5, W7):
    raise NotImplementedError("write your pallas kernel here")


# speedup vs baseline: 4.6196x; 4.6196x over previous
import functools

import jax
import jax.numpy as jnp
from jax.experimental import pallas as pl
from jax.experimental.pallas import tpu as pltpu
from jax.experimental.pallas import tpu_sc as plsc


def _smooth_body(w3_ref, w5_ref, w7_ref, out_ref):
    for k, ref, col in ((3, w3_ref, 0), (5, w5_ref, 64), (7, w7_ref, 128)):
        w = ref[...]
        p = (k - 1) // 2
        r, d = w.shape
        z = jnp.zeros((r, p), jnp.float32)
        wp = jnp.concatenate([z, w, z], axis=1)
        acc = wp[:, 0:d]
        for j in range(1, k):
            acc = acc + wp[:, j:j + d]
        out_ref[:, col:col + d] = acc * (1.0 / k)
    r, d = w3_ref.shape
    out_ref[:, 3 * d:4 * d] = jnp.zeros((r, d), jnp.float32)


def _smooth_tables(w3, w5, w7):
    v, d = w3.shape
    blk = 5000
    grid = v // blk
    return pl.pallas_call(
        _smooth_body,
        grid=(grid,),
        in_specs=[pl.BlockSpec((blk, d), lambda i: (i, 0))] * 3,
        out_specs=pl.BlockSpec((blk, 4 * d), lambda i: (i, 0)),
        out_shape=jax.ShapeDtypeStruct((v, 4 * d), jnp.float32),
    )(w3, w5, w7)


_NW = 32
_CHUNK = 400


def _sc_gather(table, idx):
    b = idx.shape[0]
    _, d = table.shape
    b_per_w = b // _NW
    n_chunks = b_per_w // _CHUNK
    mesh = plsc.VectorSubcoreMesh(core_axis_name="c", subcore_axis_name="s")

    @functools.partial(
        pl.kernel,
        out_type=jax.ShapeDtypeStruct((b, d), table.dtype),
        mesh=mesh,
        scratch_types=[
            pltpu.VMEM((_CHUNK,), jnp.int32),
            pltpu.VMEM((_CHUNK, d), jnp.float32),
            pltpu.SemaphoreType.DMA,
        ],
    )
    def gather_kernel(table_hbm, idx_hbm, out_hbm, idx_v, rows_v, sem):
        wid = jax.lax.axis_index("s") * 2 + jax.lax.axis_index("c")
        tile_base = wid * b_per_w

        @pl.loop(0, n_chunks)
        def _(c):
            base = tile_base + c * _CHUNK
            pltpu.sync_copy(idx_hbm.at[pl.ds(base, _CHUNK)], idx_v)
            pltpu.async_copy(table_hbm.at[idx_v], rows_v, sem).wait()
            pltpu.sync_copy(rows_v, out_hbm.at[pl.ds(base, _CHUNK)])

    return gather_kernel(table, idx)


def kernel(x, W3, W5, W7):
    bsz, seq = x.shape
    v, d = W3.shape
    wcat = _smooth_tables(W3, W5, W7)
    idx = x.reshape(-1).astype(jnp.int32)
    out = _sc_gather(wcat, idx)
    return out[:, :3 * d].reshape(bsz, seq, 3 * d)
